# baseline (speedup 1.0000x reference)
import functools

import jax
import jax.numpy as jnp
from jax import lax
from jax.experimental import pallas as pl
from jax.experimental.pallas import tpu as pltpu
from jax.experimental.pallas import tpu_sc as plsc

F = 26
V = 100000
D = 32
B = 16384
NC, NS = 2, 16
NW = NC * NS
RW = B // NW
EW = RW * F
IW = 128
JW = EW // IW
C = 64
E = C * F
G = E // IW
NCHUNK = RW // C

_mesh = plsc.VectorSubcoreMesh(
    core_axis_name="c", subcore_axis_name="s", num_cores=NC, num_subcores=NS
)


@functools.partial(
    pl.kernel,
    out_type=jax.ShapeDtypeStruct((B * F, D), jnp.float32),
    mesh=_mesh,
    scratch_types=[
        pltpu.VMEM((JW, IW), jnp.int32),
        pltpu.VMEM((E, D), jnp.float32),
        pltpu.SemaphoreType.DMA,
    ],
    compiler_params=pltpu.CompilerParams(use_tc_tiling_on_sc=False),
)
def _emb(tab_hbm, idx_hbm, out_hbm, idx_v, rows_v, sem):
    wid = lax.axis_index("s") * NC + lax.axis_index("c")
    lanes = lax.iota(jnp.int32, 16)
    pltpu.sync_copy(idx_hbm.at[pl.ds(wid * JW, JW)], idx_v)

    @pl.loop(0, JW * (IW // 16))
    def _adj(t):
        j = t // (IW // 16)
        s = t % (IW // 16)
        sl = (j, pl.ds(s * 16, 16))
        idx_v[sl] = idx_v[sl] + lax.rem(lanes + (j * IW + s * 16), F) * V

    @pl.loop(0, NCHUNK)
    def _chunk(g):
        cps = [
            pltpu.async_copy(
                tab_hbm.at[idx_v.at[g * G + j]],
                rows_v.at[pl.ds(j * IW, IW)],
                sem,
            )
            for j in range(G)
        ]
        for cp in cps:
            cp.wait()
        out0 = wid * EW + g * E
        pltpu.sync_copy(rows_v, out_hbm.at[pl.ds(out0, E)])


def kernel(index_list, tables):
    tab = tables.reshape(F * V, D)
    idx = index_list.astype(jnp.int32).reshape(B * F // IW, IW)
    out = _emb(tab, idx)
    return out.reshape(B, F * D)


# speedup vs baseline: 3.0660x; 3.0660x over previous
import functools

import jax
import jax.numpy as jnp
from jax import lax
from jax.experimental import pallas as pl
from jax.experimental.pallas import tpu as pltpu
from jax.experimental.pallas import tpu_sc as plsc

F = 26
V = 100000
D = 32
B = 16384
NC, NS = 2, 16
NW = NC * NS
NR = F * D
RPW = NR // NW
CHUNK = 8192
NCH = B // CHUNK

_mesh = plsc.VectorSubcoreMesh(
    core_axis_name="c", subcore_axis_name="s", num_cores=NC, num_subcores=NS
)


@functools.partial(
    pl.kernel,
    out_type=jax.ShapeDtypeStruct((NR, B), jnp.float32),
    mesh=_mesh,
    scratch_types=[
        pltpu.VMEM((V,), jnp.float32),
        pltpu.VMEM((B,), jnp.int32),
        pltpu.VMEM((CHUNK,), jnp.float32),
        pltpu.SemaphoreType.DMA,
        pltpu.SemaphoreType.DMA,
        pltpu.SemaphoreType.DMA,
    ],
    compiler_params=pltpu.CompilerParams(needs_layout_passes=False),
)
def _emb(tab, idx, out, row_v, idx_v, dst_v, sem_r, sem_i, sem_o):
    wid = lax.axis_index("s") * NC + lax.axis_index("c")
    j0 = wid * RPW

    @pl.loop(0, RPW)
    def _row(r):
        j = j0 + r
        f = j // D
        d = lax.rem(j, D)
        cp_i = pltpu.async_copy(idx.at[f], idx_v, sem_i)
        cp_r = pltpu.async_copy(tab.at[f, d, pl.ds(0, V)], row_v, sem_r)
        cp_i.wait()
        cp_r.wait()
        for h in range(NCH):
            @pl.loop(0, CHUNK // 16, unroll=4)
            def _g(t):
                iv = idx_v[pl.ds(h * CHUNK + t * 16, 16)]
                dst_v[pl.ds(t * 16, 16)] = plsc.load_gather(row_v, [iv])

            pltpu.async_copy(
                dst_v, out.at[j, pl.ds(h * CHUNK, CHUNK)], sem_o
            ).wait()


def kernel(index_list, tables):
    tab_t = tables.transpose(0, 2, 1)
    idx_t = index_list.astype(jnp.int32).T
    out_t = _emb(tab_t, idx_t)
    return out_t.T


# speedup vs baseline: 5.8250x; 1.8999x over previous
import functools

import jax
import jax.numpy as jnp
from jax import lax
from jax.experimental import pallas as pl
from jax.experimental.pallas import tpu as pltpu
from jax.experimental.pallas import tpu_sc as plsc

F = 26
V = 100000
D = 32
B = 16384
NC, NS = 2, 16
NW = NC * NS
NR = F * D
RPW = NR // NW
CHUNK = 8192
NCH = B // CHUNK

_mesh = plsc.VectorSubcoreMesh(
    core_axis_name="c", subcore_axis_name="s", num_cores=NC, num_subcores=NS
)


@functools.partial(
    pl.kernel,
    out_type=jax.ShapeDtypeStruct((NR, B), jnp.float32),
    mesh=_mesh,
    scratch_types=[
        pltpu.VMEM((V,), jnp.float32),
        pltpu.VMEM((B,), jnp.int32),
        pltpu.VMEM((CHUNK,), jnp.float32),
        pltpu.SemaphoreType.DMA,
        pltpu.SemaphoreType.DMA,
        pltpu.SemaphoreType.DMA,
    ],
    compiler_params=pltpu.CompilerParams(needs_layout_passes=False),
)
def _emb(tab, idx, out, row_v, idx_v, dst_v, sem_r, sem_i, sem_o):
    wid = lax.axis_index("s") * NC + lax.axis_index("c")
    j0 = wid * RPW

    @pl.loop(0, RPW)
    def _row(r):
        j = j0 + r
        f = j // D
        d = lax.rem(j, D)
        cp_i = pltpu.async_copy(idx.at[f], idx_v, sem_i)
        cp_r = pltpu.async_copy(tab.at[f, d, pl.ds(0, V)], row_v, sem_r)
        cp_i.wait()
        cp_r.wait()
        for h in range(NCH):
            @plsc.parallel_loop(0, CHUNK // 16, unroll=8)
            def _g(t):
                iv = idx_v[pl.ds(h * CHUNK + t * 16, 16)]
                dst_v[pl.ds(t * 16, 16)] = plsc.load_gather(row_v, [iv])

            pltpu.async_copy(
                dst_v, out.at[j, pl.ds(h * CHUNK, CHUNK)], sem_o
            ).wait()


def kernel(index_list, tables):
    tab_t = tables.transpose(0, 2, 1)
    idx_t = index_list.astype(jnp.int32).T
    out_t = _emb(tab_t, idx_t)
    return out_t.T
